# baseline (speedup 1.0000x reference)
import jax
import jax.numpy as jnp
from jax import lax
from jax.experimental import pallas as pl
from jax.experimental.pallas import tpu as pltpu
from jax.experimental.pallas import tpu_sc as plsc

B, N, F = 4, 10000, 256
E = 160000
NC, NS, L = 2, 16, 16
FH = F // NC
FGH = FH // L
ACC_ROWS = 10240
CH = 80
NB = 4
EPT = E // NS
NCHUNK = EPT // CH
ZR = ACC_ROWS // NS
DR = 80
NDRC = N // DR


def _splat(vec, lane):
    idx = jnp.full((L,), lane, dtype=jnp.int32)
    return vec.at[idx].get(mode="promise_in_bounds")


def _sc_body(xs, epr, ev, t1, t2, t3, acc, *rest):
    G = rest[0:NB]
    SR = rest[NB:2 * NB]
    VL = rest[2 * NB:3 * NB]
    SD = rest[3 * NB:4 * NB]
    GS = rest[4 * NB:5 * NB]
    SS = rest[5 * NB:6 * NB]
    IS = rest[6 * NB:7 * NB]
    zsem, dsem = rest[7 * NB:]
    c = lax.axis_index("c")
    s_ = lax.axis_index("s")

    for step in range(3):
        xin = (xs, t1, t2)[step]
        outp = (t1, t2, t3)[step]
        dbl = step > 0
        il = step == 0

        def graph_body(b, _, xin=xin, outp=outp, dbl=dbl, il=il):
            g8 = 2 * b + c

            def zz(i, _):
                for f in range(FGH):
                    G[0][i, pl.ds(f * L, L)] = jnp.zeros((L,), jnp.float32)
                return 0
            lax.fori_loop(0, CH, zz, 0)
            for j in range(ZR // CH):
                pltpu.async_copy(G[0], acc.at[pl.ds(s_ * ZR + j * CH, CH)],
                                 zsem)

            def issue_idx(ci, x):
                e0p = s_ * (2 * EPT) + ci * (2 * CH)
                e0 = s_ * EPT + ci * CH
                pltpu.async_copy(epr.at[b, pl.ds(e0p, 2 * CH)], SR[x], IS[x])
                pltpu.async_copy(ev.at[b, pl.ds(e0, CH)], VL[x], IS[x])

            def wait_idx(x):
                pltpu.make_async_copy(epr.at[b, pl.ds(0, 2 * CH)], SR[x], IS[x]).wait()
                pltpu.make_async_copy(ev.at[b, pl.ds(0, CH)], VL[x], IS[x]).wait()

            def issue_gather(x):
                if il:
                    for g in range(CH // L):
                        sl16 = pl.ds(g * L, L)
                        v = SR[x][sl16]
                        SR[x][sl16] = v + v + c
                    pltpu.async_copy(xin.at[b].at[SR[x].at[pl.ds(0, CH)]],
                                     G[x], GS[x])
                else:
                    pltpu.async_copy(xin.at[g8].at[SR[x].at[pl.ds(0, CH)]],
                                     G[x], GS[x])

            def wait_gather(x):
                src = xin.at[b] if il else xin.at[g8]
                pltpu.make_async_copy(src.at[SR[x].at[pl.ds(0, CH)]],
                                      G[x], GS[x]).wait()

            def issue_scatter(x):
                pltpu.async_copy(G[x], acc.at[SD[x]], SS[x], add=True)

            def wait_scatter(x):
                pltpu.make_async_copy(G[x], acc.at[SD[x]], SS[x]).wait()

            for x in range(NB):
                issue_idx(x, x)
            for x in range(1, NB - 1):
                wait_idx(x)
                issue_gather(x)
            for j in range(ZR // CH):
                pltpu.make_async_copy(G[0], acc.at[pl.ds(0, CH)], zsem).wait()
            wait_idx(0)
            issue_gather(0)
            plsc.subcore_barrier()

            def scale(x, dbl=dbl):
                def grp(g, _):
                    sl16 = pl.ds(g * L, L)
                    vgrp = VL[x][sl16]
                    if dbl:
                        vgrp = vgrp + vgrp
                    SD[x][sl16] = SR[x][pl.ds(CH + g * L, L)]
                    for e in range(L):
                        r = g * L + e
                        vs = _splat(vgrp, e)
                        for f in range(FGH):
                            slf = pl.ds(f * L, L)
                            G[x][r, slf] = G[x][r, slf] * vs
                    return 0
                lax.fori_loop(0, CH // L, grp, 0)

            def process(ci, x):
                wait_gather(x)
                scale(x)
                issue_scatter(x)
                @pl.when(ci + NB < NCHUNK)
                def _():
                    issue_idx(ci + NB, x)
                @pl.when(ci + NB - 1 < NCHUNK)
                def _():
                    y = (x + NB - 1) % NB
                    @pl.when(ci >= 1)
                    def _():
                        wait_scatter(y)
                    wait_idx(y)
                    issue_gather(y)

            @pl.loop(0, NCHUNK - 1, step=NB)
            def _(ci0):
                for x in range(NB):
                    process(ci0 + x, x)

            process(NCHUNK - 1, 0)
            for x in range(NB):
                wait_scatter((x + 1) % NB)
            plsc.subcore_barrier()

            for j in range(-(-NDRC // NS)):
                cd = s_ + NS * j
                @pl.when(cd < NDRC)
                def _():
                    pltpu.async_copy(acc.at[pl.ds(cd * DR, DR)],
                                     outp.at[g8, pl.ds(cd * DR, DR)], dsem)
            for j in range(-(-NDRC // NS)):
                cd = s_ + NS * j
                @pl.when(cd < NDRC)
                def _():
                    pltpu.make_async_copy(acc.at[pl.ds(0, DR)],
                                          outp.at[g8, pl.ds(0, DR)],
                                          dsem).wait()
            plsc.subcore_barrier()
            return 0

        lax.fori_loop(0, B, graph_body, 0)


def _make_sc():
    mesh = plsc.VectorSubcoreMesh(core_axis_name="c", subcore_axis_name="s",
                                  num_cores=NC, num_subcores=NS)
    tshape = jax.ShapeDtypeStruct((2 * B, N, FH), jnp.float32)
    scratch = (
        [pltpu.VMEM_SHARED((ACC_ROWS, FH), jnp.float32)]
        + [pltpu.VMEM((CH, FH), jnp.float32) for _ in range(NB)]
        + [pltpu.VMEM((2 * CH,), jnp.int32) for _ in range(NB)]
        + [pltpu.VMEM((CH,), jnp.float32) for _ in range(NB)]
        + [pltpu.VMEM((CH,), jnp.int32) for _ in range(NB)]
        + [pltpu.SemaphoreType.DMA for _ in range(3 * NB)]
        + [pltpu.SemaphoreType.DMA, pltpu.SemaphoreType.DMA]
    )
    return pl.kernel(
        _sc_body,
        out_type=(tshape, tshape, tshape),
        mesh=mesh,
        scratch_types=scratch,
        compiler_params=pltpu.CompilerParams(use_tc_tiling_on_sc=False),
    )


_sc_spmm = _make_sc()


RB = 200
GRID = N // RB


def _mix_body(x_ref, u1_ref, u2_ref, u3_ref, w_ref, b_ref, o_ref):
    bias = b_ref[0, 0]
    ys = []
    for t in (x_ref, u1_ref, u2_ref, u3_ref):
        acc = jnp.zeros((RB, F), jnp.float32)
        for k in range(B):
            wk = w_ref[k]
            if t is x_ref:
                acc = acc + jnp.dot(t[k], wk,
                                    preferred_element_type=jnp.float32)
            else:
                acc = acc + jnp.dot(t[2 * k], wk[:FH],
                                    preferred_element_type=jnp.float32)
                acc = acc + jnp.dot(t[2 * k + 1], wk[FH:],
                                    preferred_element_type=jnp.float32)
        ys.append(acc)
    y0, y1, y2, y3 = ys
    o_ref[:, pl.ds(0, F)] = jnp.maximum(y0 + bias, 0.0)
    o_ref[:, pl.ds(F, F)] = jnp.maximum(y1 + bias, 0.0)
    o_ref[:, pl.ds(2 * F, F)] = jnp.maximum(y2 - y0 + bias, 0.0)
    o_ref[:, pl.ds(3 * F, F)] = jnp.maximum(y3 - 3.0 * y1 + bias, 0.0)


def _mix(x, u1, u2, u3, W, bias):
    xspec = pl.BlockSpec((B, RB, F), lambda i: (0, i, 0))
    tspec = pl.BlockSpec((2 * B, RB, FH), lambda i: (0, i, 0))
    out = pl.pallas_call(
        _mix_body,
        grid=(GRID,),
        in_specs=[xspec, tspec, tspec, tspec,
                  pl.BlockSpec((B, F, F), lambda i: (0, 0, 0)),
                  pl.BlockSpec((1, 1, F), lambda i: (0, 0, 0))],
        out_specs=pl.BlockSpec((RB, 4 * F), lambda i: (i, 0)),
        out_shape=jax.ShapeDtypeStruct((N, 4 * F), jnp.float32),
    )(x, u1, u2, u3, W, bias)
    return out.reshape(B, N, F)


def kernel(x, edge_index, edge_vals, W, bias):
    ei = edge_index.astype(jnp.int32)
    esrc, edst = ei[:, 0], ei[:, 1]
    epr = jnp.stack([esrc.reshape(B, NS, NCHUNK, CH),
                     edst.reshape(B, NS, NCHUNK, CH)], axis=3)
    epr = epr.reshape(B, 2 * E)
    ev = edge_vals.astype(jnp.float32)
    x = x.astype(jnp.float32)
    xs = x.reshape(B, NC * N, FH)
    u1, u2, u3 = _sc_spmm(xs, epr, ev)
    return _mix(x, u1, u2, u3, W, bias)


# speedup vs baseline: 1.0249x; 1.0249x over previous
import jax
import jax.numpy as jnp
from jax import lax
from jax.experimental import pallas as pl
from jax.experimental.pallas import tpu as pltpu
from jax.experimental.pallas import tpu_sc as plsc

B, N, F = 4, 10000, 256
E = 160000
NC, NS, L = 2, 16, 16
FH = F // NC
FGH = FH // L
ACC_ROWS = 10240
CH = 80
NB = 4
EPT = E // NS
NCHUNK = EPT // CH
ZR = ACC_ROWS // NS
DR = 80
NDRC = N // DR


def _splat(vec, lane):
    idx = jnp.full((L,), lane, dtype=jnp.int32)
    return vec.at[idx].get(mode="promise_in_bounds")


def _sc_body(xs, esrc, edst, ev, t1, t2, t3, acc, *rest):
    G = rest[0:NB]
    SR = rest[NB:2 * NB]
    DS = rest[2 * NB:3 * NB]
    VL = rest[3 * NB:4 * NB]
    SD = rest[4 * NB:5 * NB]
    GS = rest[5 * NB:6 * NB]
    SS = rest[6 * NB:7 * NB]
    IS = rest[7 * NB:8 * NB]
    zsem, dsem = rest[8 * NB:]
    c = lax.axis_index("c")
    s_ = lax.axis_index("s")

    for step in range(3):
        xin = (xs, t1, t2)[step]
        outp = (t1, t2, t3)[step]
        dbl = step > 0
        il = step == 0

        def graph_body(b, _, xin=xin, outp=outp, dbl=dbl, il=il):
            g8 = 2 * b + c

            def zz(i, _):
                for f in range(FGH):
                    G[0][i, pl.ds(f * L, L)] = jnp.zeros((L,), jnp.float32)
                return 0
            lax.fori_loop(0, CH, zz, 0)
            for j in range(ZR // CH):
                pltpu.async_copy(G[0], acc.at[pl.ds(s_ * ZR + j * CH, CH)],
                                 zsem)

            def issue_idx(ci, x):
                e0 = s_ * EPT + ci * CH
                pltpu.async_copy(esrc.at[b, pl.ds(e0, CH)], SR[x], IS[x])
                pltpu.async_copy(edst.at[b, pl.ds(e0, CH)], DS[x], IS[x])
                pltpu.async_copy(ev.at[b, pl.ds(e0, CH)], VL[x], IS[x])

            def wait_idx(x):
                pltpu.make_async_copy(esrc.at[b, pl.ds(0, CH)], SR[x], IS[x]).wait()
                pltpu.make_async_copy(edst.at[b, pl.ds(0, CH)], DS[x], IS[x]).wait()
                pltpu.make_async_copy(ev.at[b, pl.ds(0, CH)], VL[x], IS[x]).wait()

            def issue_gather(x):
                if il:
                    for g in range(CH // L):
                        sl16 = pl.ds(g * L, L)
                        v = SR[x][sl16]
                        SR[x][sl16] = v + v + c
                    pltpu.async_copy(xin.at[b].at[SR[x]], G[x], GS[x])
                else:
                    pltpu.async_copy(xin.at[g8].at[SR[x]], G[x], GS[x])

            def wait_gather(x):
                src = xin.at[b] if il else xin.at[g8]
                pltpu.make_async_copy(src.at[SR[x]], G[x], GS[x]).wait()

            def issue_scatter(x):
                pltpu.async_copy(G[x], acc.at[SD[x]], SS[x], add=True)

            def wait_scatter(x):
                pltpu.make_async_copy(G[x], acc.at[SD[x]], SS[x]).wait()

            for x in range(NB):
                issue_idx(x, x)
            for x in range(1, NB - 1):
                wait_idx(x)
                issue_gather(x)
            for j in range(ZR // CH):
                pltpu.make_async_copy(G[0], acc.at[pl.ds(0, CH)], zsem).wait()
            wait_idx(0)
            issue_gather(0)
            plsc.subcore_barrier()

            def scale(x, dbl=dbl):
                def grp(g, _):
                    sl16 = pl.ds(g * L, L)
                    vgrp = VL[x][sl16]
                    if dbl:
                        vgrp = vgrp + vgrp
                    SD[x][sl16] = DS[x][sl16]
                    for e in range(L):
                        r = g * L + e
                        vs = _splat(vgrp, e)
                        for f in range(FGH):
                            slf = pl.ds(f * L, L)
                            G[x][r, slf] = G[x][r, slf] * vs
                    return 0
                lax.fori_loop(0, CH // L, grp, 0)

            def process(ci, x):
                wait_gather(x)
                scale(x)
                issue_scatter(x)
                @pl.when(ci + NB < NCHUNK)
                def _():
                    issue_idx(ci + NB, x)
                @pl.when(ci + NB - 1 < NCHUNK)
                def _():
                    y = (x + NB - 1) % NB
                    @pl.when(ci >= 1)
                    def _():
                        wait_scatter(y)
                    wait_idx(y)
                    issue_gather(y)

            @pl.loop(0, NCHUNK - 1, step=NB)
            def _(ci0):
                for x in range(NB):
                    process(ci0 + x, x)

            process(NCHUNK - 1, 0)
            for x in range(NB):
                wait_scatter((x + 1) % NB)
            plsc.subcore_barrier()

            for j in range(-(-NDRC // NS)):
                cd = s_ + NS * j
                @pl.when(cd < NDRC)
                def _():
                    pltpu.async_copy(acc.at[pl.ds(cd * DR, DR)],
                                     outp.at[g8, pl.ds(cd * DR, DR)], dsem)
            for j in range(-(-NDRC // NS)):
                cd = s_ + NS * j
                @pl.when(cd < NDRC)
                def _():
                    pltpu.make_async_copy(acc.at[pl.ds(0, DR)],
                                          outp.at[g8, pl.ds(0, DR)],
                                          dsem).wait()
            plsc.subcore_barrier()
            return 0

        lax.fori_loop(0, B, graph_body, 0)


def _make_sc():
    mesh = plsc.VectorSubcoreMesh(core_axis_name="c", subcore_axis_name="s",
                                  num_cores=NC, num_subcores=NS)
    tshape = jax.ShapeDtypeStruct((2 * B, N, FH), jnp.float32)
    scratch = (
        [pltpu.VMEM_SHARED((ACC_ROWS, FH), jnp.float32)]
        + [pltpu.VMEM((CH, FH), jnp.float32) for _ in range(NB)]
        + [pltpu.VMEM((CH,), jnp.int32) for _ in range(NB)]
        + [pltpu.VMEM((CH,), jnp.int32) for _ in range(NB)]
        + [pltpu.VMEM((CH,), jnp.float32) for _ in range(NB)]
        + [pltpu.VMEM((CH,), jnp.int32) for _ in range(NB)]
        + [pltpu.SemaphoreType.DMA for _ in range(3 * NB)]
        + [pltpu.SemaphoreType.DMA, pltpu.SemaphoreType.DMA]
    )
    return pl.kernel(
        _sc_body,
        out_type=(tshape, tshape, tshape),
        mesh=mesh,
        scratch_types=scratch,
        compiler_params=pltpu.CompilerParams(use_tc_tiling_on_sc=False),
    )


_sc_spmm = _make_sc()


RB = 200
GRID = N // RB


def _mix_body(x_ref, u1_ref, u2_ref, u3_ref, w_ref, b_ref, o_ref):
    bias = b_ref[0, 0]
    ys = []
    for t in (x_ref, u1_ref, u2_ref, u3_ref):
        acc = jnp.zeros((RB, F), jnp.float32)
        for k in range(B):
            wk = w_ref[k]
            if t is x_ref:
                acc = acc + jnp.dot(t[k], wk,
                                    preferred_element_type=jnp.float32)
            else:
                acc = acc + jnp.dot(t[2 * k], wk[:FH],
                                    preferred_element_type=jnp.float32)
                acc = acc + jnp.dot(t[2 * k + 1], wk[FH:],
                                    preferred_element_type=jnp.float32)
        ys.append(acc)
    y0, y1, y2, y3 = ys
    o_ref[:, pl.ds(0, F)] = jnp.maximum(y0 + bias, 0.0)
    o_ref[:, pl.ds(F, F)] = jnp.maximum(y1 + bias, 0.0)
    o_ref[:, pl.ds(2 * F, F)] = jnp.maximum(y2 - y0 + bias, 0.0)
    o_ref[:, pl.ds(3 * F, F)] = jnp.maximum(y3 - 3.0 * y1 + bias, 0.0)


def _mix(x, u1, u2, u3, W, bias):
    xspec = pl.BlockSpec((B, RB, F), lambda i: (0, i, 0))
    tspec = pl.BlockSpec((2 * B, RB, FH), lambda i: (0, i, 0))
    out = pl.pallas_call(
        _mix_body,
        grid=(GRID,),
        in_specs=[xspec, tspec, tspec, tspec,
                  pl.BlockSpec((B, F, F), lambda i: (0, 0, 0)),
                  pl.BlockSpec((1, 1, F), lambda i: (0, 0, 0))],
        out_specs=pl.BlockSpec((RB, 4 * F), lambda i: (i, 0)),
        out_shape=jax.ShapeDtypeStruct((N, 4 * F), jnp.float32),
    )(x, u1, u2, u3, W, bias)
    return out.reshape(B, N, F)


def kernel(x, edge_index, edge_vals, W, bias):
    ei = edge_index.astype(jnp.int32)
    esrc, edst = ei[:, 0], ei[:, 1]
    ev = edge_vals.astype(jnp.float32)
    x = x.astype(jnp.float32)
    xs = x.reshape(B, NC * N, FH)
    u1, u2, u3 = _sc_spmm(xs, esrc, edst, ev)
    return _mix(x, u1, u2, u3, W, bias)


# speedup vs baseline: 1.0353x; 1.0102x over previous
import jax
import jax.numpy as jnp
from jax import lax
from jax.experimental import pallas as pl
from jax.experimental.pallas import tpu as pltpu
from jax.experimental.pallas import tpu_sc as plsc

B, N, F = 4, 10000, 256
E = 160000
NC, NS, L = 2, 16, 16
FH = F // NC
FGH = FH // L
ACC_ROWS = 10240
CH = 80
NB = 4
EPT = E // NS
NCHUNK = EPT // CH
ZR = ACC_ROWS // NS
DR = 80
NDRC = N // DR


def _splat(vec, lane):
    idx = jnp.full((L,), lane, dtype=jnp.int32)
    return vec.at[idx].get(mode="promise_in_bounds")


def _sc_body(xs, esrc, edst, ev, t1, t2, t3, acc, *rest):
    G = rest[0:NB]
    SR = rest[NB:2 * NB]
    DS = rest[2 * NB:3 * NB]
    VL = rest[3 * NB:4 * NB]
    SD = rest[4 * NB:5 * NB]
    GS = rest[5 * NB:6 * NB]
    SS = rest[6 * NB:7 * NB]
    IS = rest[7 * NB:8 * NB]
    zsem, dsem = rest[8 * NB:]
    c = lax.axis_index("c")
    s_ = lax.axis_index("s")

    for step in range(3):
        xin = (xs, t1, t2)[step]
        outp = (t1, t2, t3)[step]
        dbl = step > 0
        il = step == 0

        def graph_body(b, _, xin=xin, outp=outp, dbl=dbl, il=il):
            g8 = 2 * b + c

            def zz(i, _):
                for f in range(FGH):
                    G[0][i, pl.ds(f * L, L)] = jnp.zeros((L,), jnp.float32)
                return 0
            lax.fori_loop(0, CH, zz, 0)
            for j in range(ZR // CH):
                pltpu.async_copy(G[0], acc.at[pl.ds(s_ * ZR + j * CH, CH)],
                                 zsem)

            def issue_idx(ci, x):
                e0 = s_ * EPT + ci * CH
                pltpu.async_copy(esrc.at[b, pl.ds(e0, CH)], SR[x], IS[x])
                pltpu.async_copy(edst.at[b, pl.ds(e0, CH)], DS[x], IS[x])
                pltpu.async_copy(ev.at[b, pl.ds(e0, CH)], VL[x], IS[x])

            def wait_idx(x):
                pltpu.make_async_copy(esrc.at[b, pl.ds(0, CH)], SR[x], IS[x]).wait()
                pltpu.make_async_copy(edst.at[b, pl.ds(0, CH)], DS[x], IS[x]).wait()
                pltpu.make_async_copy(ev.at[b, pl.ds(0, CH)], VL[x], IS[x]).wait()

            def issue_gather(x):
                if il:
                    for g in range(CH // L):
                        sl16 = pl.ds(g * L, L)
                        v = SR[x][sl16]
                        SR[x][sl16] = v + v + c
                    pltpu.async_copy(xin.at[b].at[SR[x]], G[x], GS[x])
                else:
                    pltpu.async_copy(xin.at[g8].at[SR[x]], G[x], GS[x])

            def wait_gather(x):
                src = xin.at[b] if il else xin.at[g8]
                pltpu.make_async_copy(src.at[SR[x]], G[x], GS[x]).wait()

            def issue_scatter(x):
                pltpu.async_copy(G[x], acc.at[SD[x]], SS[x], add=True)

            def wait_scatter(x):
                pltpu.make_async_copy(G[x], acc.at[SD[x]], SS[x]).wait()

            for x in range(NB):
                issue_idx(x, x)
            for x in range(1, NB - 1):
                wait_idx(x)
                issue_gather(x)
            for j in range(ZR // CH):
                pltpu.make_async_copy(G[0], acc.at[pl.ds(0, CH)], zsem).wait()
            wait_idx(0)
            issue_gather(0)
            plsc.subcore_barrier()

            def scale(x, dbl=dbl):
                def grp(g, _):
                    sl16 = pl.ds(g * L, L)
                    vgrp = VL[x][sl16]
                    if dbl:
                        vgrp = vgrp + vgrp
                    SD[x][sl16] = DS[x][sl16]
                    for e in range(L):
                        r = g * L + e
                        vs = _splat(vgrp, e)
                        for f in range(FGH):
                            slf = pl.ds(f * L, L)
                            G[x][r, slf] = G[x][r, slf] * vs
                    return 0
                lax.fori_loop(0, CH // L, grp, 0)

            def process(ci, x):
                wait_gather(x)
                scale(x)
                issue_scatter(x)
                @pl.when(ci + NB < NCHUNK)
                def _():
                    issue_idx(ci + NB, x)
                @pl.when(ci + NB - 1 < NCHUNK)
                def _():
                    y = (x + NB - 1) % NB
                    @pl.when(ci >= 1)
                    def _():
                        wait_scatter(y)
                    wait_idx(y)
                    issue_gather(y)

            @pl.loop(0, NCHUNK - 1, step=NB)
            def _(ci0):
                for x in range(NB):
                    process(ci0 + x, x)

            process(NCHUNK - 1, 0)
            for x in range(NB):
                wait_scatter((x + 1) % NB)
            plsc.subcore_barrier()

            for j in range(-(-NDRC // NS)):
                cd = s_ + NS * j
                @pl.when(cd < NDRC)
                def _():
                    pltpu.async_copy(acc.at[pl.ds(cd * DR, DR)],
                                     outp.at[g8, pl.ds(cd * DR, DR)], dsem)
            for j in range(-(-NDRC // NS)):
                cd = s_ + NS * j
                @pl.when(cd < NDRC)
                def _():
                    pltpu.make_async_copy(acc.at[pl.ds(0, DR)],
                                          outp.at[g8, pl.ds(0, DR)],
                                          dsem).wait()
            plsc.subcore_barrier()
            return 0

        lax.fori_loop(0, B, graph_body, 0)


def _make_sc():
    mesh = plsc.VectorSubcoreMesh(core_axis_name="c", subcore_axis_name="s",
                                  num_cores=NC, num_subcores=NS)
    tshape = jax.ShapeDtypeStruct((2 * B, N, FH), jnp.float32)
    scratch = (
        [pltpu.VMEM_SHARED((ACC_ROWS, FH), jnp.float32)]
        + [pltpu.VMEM((CH, FH), jnp.float32) for _ in range(NB)]
        + [pltpu.VMEM((CH,), jnp.int32) for _ in range(NB)]
        + [pltpu.VMEM((CH,), jnp.int32) for _ in range(NB)]
        + [pltpu.VMEM((CH,), jnp.float32) for _ in range(NB)]
        + [pltpu.VMEM((CH,), jnp.int32) for _ in range(NB)]
        + [pltpu.SemaphoreType.DMA for _ in range(3 * NB)]
        + [pltpu.SemaphoreType.DMA, pltpu.SemaphoreType.DMA]
    )
    return pl.kernel(
        _sc_body,
        out_type=(tshape, tshape, tshape),
        mesh=mesh,
        scratch_types=scratch,
        compiler_params=pltpu.CompilerParams(use_tc_tiling_on_sc=False),
    )


_sc_spmm = _make_sc()


RB = 400
GRID = N // RB


def _mix_body(x_ref, u1_ref, u2_ref, u3_ref, w_ref, b_ref, o_ref):
    bias = b_ref[0, 0]
    ys = []
    for t in (x_ref, u1_ref, u2_ref, u3_ref):
        acc = jnp.zeros((RB, F), jnp.float32)
        for k in range(B):
            wk = w_ref[k]
            if t is x_ref:
                acc = acc + jnp.dot(t[k], wk,
                                    preferred_element_type=jnp.float32)
            else:
                acc = acc + jnp.dot(t[2 * k], wk[:FH],
                                    preferred_element_type=jnp.float32)
                acc = acc + jnp.dot(t[2 * k + 1], wk[FH:],
                                    preferred_element_type=jnp.float32)
        ys.append(acc)
    y0, y1, y2, y3 = ys
    o_ref[:, pl.ds(0, F)] = jnp.maximum(y0 + bias, 0.0)
    o_ref[:, pl.ds(F, F)] = jnp.maximum(y1 + bias, 0.0)
    o_ref[:, pl.ds(2 * F, F)] = jnp.maximum(y2 - y0 + bias, 0.0)
    o_ref[:, pl.ds(3 * F, F)] = jnp.maximum(y3 - 3.0 * y1 + bias, 0.0)


def _mix(x, u1, u2, u3, W, bias):
    xspec = pl.BlockSpec((B, RB, F), lambda i: (0, i, 0))
    tspec = pl.BlockSpec((2 * B, RB, FH), lambda i: (0, i, 0))
    out = pl.pallas_call(
        _mix_body,
        grid=(GRID,),
        in_specs=[xspec, tspec, tspec, tspec,
                  pl.BlockSpec((B, F, F), lambda i: (0, 0, 0)),
                  pl.BlockSpec((1, 1, F), lambda i: (0, 0, 0))],
        out_specs=pl.BlockSpec((RB, 4 * F), lambda i: (i, 0)),
        out_shape=jax.ShapeDtypeStruct((N, 4 * F), jnp.float32),
    )(x, u1, u2, u3, W, bias)
    return out.reshape(B, N, F)


def kernel(x, edge_index, edge_vals, W, bias):
    ei = edge_index.astype(jnp.int32)
    esrc, edst = ei[:, 0], ei[:, 1]
    ev = edge_vals.astype(jnp.float32)
    x = x.astype(jnp.float32)
    xs = x.reshape(B, NC * N, FH)
    u1, u2, u3 = _sc_spmm(xs, esrc, edst, ev)
    return _mix(x, u1, u2, u3, W, bias)


# speedup vs baseline: 1.0380x; 1.0026x over previous
import jax
import jax.numpy as jnp
from jax import lax
from jax.experimental import pallas as pl
from jax.experimental.pallas import tpu as pltpu
from jax.experimental.pallas import tpu_sc as plsc

B, N, F = 4, 10000, 256
E = 160000
NC, NS, L = 2, 16, 16
FH = F // NC
FGH = FH // L
ACC_ROWS = 10240
CH = 80
NB = 4
EPT = E // NS
NCHUNK = EPT // CH
ZR = ACC_ROWS // NS
DR = 80
NDRC = N // DR


def _splat(vec, lane):
    idx = jnp.full((L,), lane, dtype=jnp.int32)
    return vec.at[idx].get(mode="promise_in_bounds")


def _sc_body(xs, esrc, edst, ev, t1, t2, t3, acc, *rest):
    G = rest[0:NB]
    SR = rest[NB:2 * NB]
    DS = rest[2 * NB:3 * NB]
    VL = rest[3 * NB:4 * NB]
    SD = rest[4 * NB:5 * NB]
    GS = rest[5 * NB:6 * NB]
    SS = rest[6 * NB:7 * NB]
    IS = rest[7 * NB:8 * NB]
    zsem, dsem = rest[8 * NB:]
    c = lax.axis_index("c")
    s_ = lax.axis_index("s")

    for step in range(3):
        xin = (xs, t1, t2)[step]
        outp = (t1, t2, t3)[step]
        dbl = step > 0
        il = step == 0

        def graph_body(b, _, xin=xin, outp=outp, dbl=dbl, il=il):
            g8 = 2 * b + c

            def zz(i, _):
                for f in range(FGH):
                    G[0][i, pl.ds(f * L, L)] = jnp.zeros((L,), jnp.float32)
                return 0
            lax.fori_loop(0, CH, zz, 0)
            for j in range(ZR // CH):
                pltpu.async_copy(G[0], acc.at[pl.ds(s_ * ZR + j * CH, CH)],
                                 zsem)

            def issue_idx(ci, x):
                e0 = s_ * EPT + ci * CH
                pltpu.async_copy(esrc.at[b, pl.ds(e0, CH)], SR[x], IS[x])
                pltpu.async_copy(edst.at[b, pl.ds(e0, CH)], DS[x], IS[x])
                pltpu.async_copy(ev.at[b, pl.ds(e0, CH)], VL[x], IS[x])

            def wait_idx(x):
                pltpu.make_async_copy(esrc.at[b, pl.ds(0, CH)], SR[x], IS[x]).wait()
                pltpu.make_async_copy(edst.at[b, pl.ds(0, CH)], DS[x], IS[x]).wait()
                pltpu.make_async_copy(ev.at[b, pl.ds(0, CH)], VL[x], IS[x]).wait()

            def issue_gather(x):
                if il:
                    for g in range(CH // L):
                        sl16 = pl.ds(g * L, L)
                        v = SR[x][sl16]
                        SR[x][sl16] = v + v + c
                    pltpu.async_copy(xin.at[b].at[SR[x]], G[x], GS[x])
                else:
                    pltpu.async_copy(xin.at[g8].at[SR[x]], G[x], GS[x])

            def wait_gather(x):
                src = xin.at[b] if il else xin.at[g8]
                pltpu.make_async_copy(src.at[SR[x]], G[x], GS[x]).wait()

            def issue_scatter(x):
                pltpu.async_copy(G[x], acc.at[SD[x]], SS[x], add=True)

            def wait_scatter(x):
                pltpu.make_async_copy(G[x], acc.at[SD[x]], SS[x]).wait()

            for x in range(NB):
                issue_idx(x, x)
            for x in range(1, NB - 1):
                wait_idx(x)
                issue_gather(x)
            for j in range(ZR // CH):
                pltpu.make_async_copy(G[0], acc.at[pl.ds(0, CH)], zsem).wait()
            wait_idx(0)
            issue_gather(0)
            plsc.subcore_barrier()

            def scale(x, dbl=dbl):
                def grp(g, _):
                    sl16 = pl.ds(g * L, L)
                    vgrp = VL[x][sl16]
                    if dbl:
                        vgrp = vgrp + vgrp
                    SD[x][sl16] = DS[x][sl16]
                    for e in range(L):
                        r = g * L + e
                        vs = _splat(vgrp, e)
                        for f in range(FGH):
                            slf = pl.ds(f * L, L)
                            G[x][r, slf] = G[x][r, slf] * vs
                    return 0
                lax.fori_loop(0, CH // L, grp, 0)

            def process(ci, x):
                wait_gather(x)
                scale(x)
                issue_scatter(x)
                @pl.when(ci + NB < NCHUNK)
                def _():
                    issue_idx(ci + NB, x)
                @pl.when(ci + NB - 1 < NCHUNK)
                def _():
                    y = (x + NB - 1) % NB
                    @pl.when(ci >= 1)
                    def _():
                        wait_scatter(y)
                    wait_idx(y)
                    issue_gather(y)

            @pl.loop(0, NCHUNK - 1, step=NB)
            def _(ci0):
                for x in range(NB):
                    process(ci0 + x, x)

            process(NCHUNK - 1, 0)
            for x in range(NB):
                wait_scatter((x + 1) % NB)
            plsc.subcore_barrier()

            for j in range(-(-NDRC // NS)):
                cd = s_ + NS * j
                @pl.when(cd < NDRC)
                def _():
                    pltpu.async_copy(acc.at[pl.ds(cd * DR, DR)],
                                     outp.at[g8, pl.ds(cd * DR, DR)], dsem)
            for j in range(-(-NDRC // NS)):
                cd = s_ + NS * j
                @pl.when(cd < NDRC)
                def _():
                    pltpu.make_async_copy(acc.at[pl.ds(0, DR)],
                                          outp.at[g8, pl.ds(0, DR)],
                                          dsem).wait()
            plsc.subcore_barrier()
            return 0

        lax.fori_loop(0, B, graph_body, 0)


def _make_sc():
    mesh = plsc.VectorSubcoreMesh(core_axis_name="c", subcore_axis_name="s",
                                  num_cores=NC, num_subcores=NS)
    tshape = jax.ShapeDtypeStruct((2 * B, N, FH), jnp.float32)
    scratch = (
        [pltpu.VMEM_SHARED((ACC_ROWS, FH), jnp.float32)]
        + [pltpu.VMEM((CH, FH), jnp.float32) for _ in range(NB)]
        + [pltpu.VMEM((CH,), jnp.int32) for _ in range(NB)]
        + [pltpu.VMEM((CH,), jnp.int32) for _ in range(NB)]
        + [pltpu.VMEM((CH,), jnp.float32) for _ in range(NB)]
        + [pltpu.VMEM((CH,), jnp.int32) for _ in range(NB)]
        + [pltpu.SemaphoreType.DMA for _ in range(3 * NB)]
        + [pltpu.SemaphoreType.DMA, pltpu.SemaphoreType.DMA]
    )
    return pl.kernel(
        _sc_body,
        out_type=(tshape, tshape, tshape),
        mesh=mesh,
        scratch_types=scratch,
        compiler_params=pltpu.CompilerParams(use_tc_tiling_on_sc=False),
    )


_sc_spmm = _make_sc()


RB = 1000
GRID = N // RB


def _mix_body(x_ref, u1_ref, u2_ref, u3_ref, w_ref, b_ref, o_ref):
    bias = b_ref[0, 0]
    ys = []
    for t in (x_ref, u1_ref, u2_ref, u3_ref):
        acc = jnp.zeros((RB, F), jnp.float32)
        for k in range(B):
            wk = w_ref[k]
            if t is x_ref:
                acc = acc + jnp.dot(t[k], wk,
                                    preferred_element_type=jnp.float32)
            else:
                acc = acc + jnp.dot(t[2 * k], wk[:FH],
                                    preferred_element_type=jnp.float32)
                acc = acc + jnp.dot(t[2 * k + 1], wk[FH:],
                                    preferred_element_type=jnp.float32)
        ys.append(acc)
    y0, y1, y2, y3 = ys
    o_ref[:, pl.ds(0, F)] = jnp.maximum(y0 + bias, 0.0)
    o_ref[:, pl.ds(F, F)] = jnp.maximum(y1 + bias, 0.0)
    o_ref[:, pl.ds(2 * F, F)] = jnp.maximum(y2 - y0 + bias, 0.0)
    o_ref[:, pl.ds(3 * F, F)] = jnp.maximum(y3 - 3.0 * y1 + bias, 0.0)


def _mix(x, u1, u2, u3, W, bias):
    xspec = pl.BlockSpec((B, RB, F), lambda i: (0, i, 0))
    tspec = pl.BlockSpec((2 * B, RB, FH), lambda i: (0, i, 0))
    out = pl.pallas_call(
        _mix_body,
        grid=(GRID,),
        in_specs=[xspec, tspec, tspec, tspec,
                  pl.BlockSpec((B, F, F), lambda i: (0, 0, 0)),
                  pl.BlockSpec((1, 1, F), lambda i: (0, 0, 0))],
        out_specs=pl.BlockSpec((RB, 4 * F), lambda i: (i, 0)),
        out_shape=jax.ShapeDtypeStruct((N, 4 * F), jnp.float32),
    )(x, u1, u2, u3, W, bias)
    return out.reshape(B, N, F)


def kernel(x, edge_index, edge_vals, W, bias):
    ei = edge_index.astype(jnp.int32)
    esrc, edst = ei[:, 0], ei[:, 1]
    ev = edge_vals.astype(jnp.float32)
    x = x.astype(jnp.float32)
    xs = x.reshape(B, NC * N, FH)
    u1, u2, u3 = _sc_spmm(xs, esrc, edst, ev)
    return _mix(x, u1, u2, u3, W, bias)
